# baseline (speedup 1.0000x reference)
import functools

import jax
import jax.numpy as jnp
from jax.experimental import pallas as pl
from jax.experimental.pallas import tpu as pltpu

_NUM_EXPERTS = 16
_SUPER = 1024
_NSUB = 4
_SUB = _SUPER // _NSUB
_NSLOT = 4


def _copy(h_hbm, buf, sem, block, slot, s):
    return pltpu.make_async_copy(
        h_hbm.at[pl.ds(block * _SUPER + s * _SUB, _SUB), :],
        buf.at[slot, pl.ds(s * _SUB, _SUB), :],
        sem.at[slot, s],
    )


def _issue(h_hbm, buf, sem, block, slot):
    for s in range(_NSUB):
        _copy(h_hbm, buf, sem, block, slot, s).start()


def _wait(h_hbm, buf, sem, block, slot):
    for s in range(_NSUB):
        _copy(h_hbm, buf, sem, block, slot, s).wait()


def _gate_kernel(h_hbm, wt_ref, b_ref, out_ref, buf, sem):
    i = pl.program_id(0)
    nblocks = pl.num_programs(0)

    @pl.when(i == 0)
    def _warmup():
        for k in range(_NSLOT):
            _issue(h_hbm, buf, sem, k, k)

    slot = jax.lax.rem(i, _NSLOT)
    _wait(h_hbm, buf, sem, i, slot)

    g = jax.lax.dot_general(
        buf[slot], wt_ref[...],
        dimension_numbers=(((1,), (1,)), ((), ())),
        preferred_element_type=jnp.float32,
    )
    g = g + b_ref[...]
    idx = jax.lax.broadcasted_iota(jnp.int32, g.shape, 1).astype(jnp.float32)
    ne_f = jnp.float32(_NUM_EXPERTS)
    m1 = jnp.max(g, axis=1, keepdims=True)
    i1 = jnp.min(jnp.where(g == m1, idx, ne_f), axis=1, keepdims=True)
    g2 = jnp.where(idx == i1, -jnp.inf, g)
    m2 = jnp.max(g2, axis=1, keepdims=True)
    i2 = jnp.min(jnp.where(g2 == m2, idx, ne_f), axis=1, keepdims=True)
    mask = (idx == i1) | (idx == i2)
    e = jnp.exp(g - m1)
    denom = 1.0 + jnp.exp(m2 - m1)
    out_ref[...] = jnp.where(mask, e / denom, 0.0)

    @pl.when(i + _NSLOT < nblocks)
    def _refill():
        _issue(h_hbm, buf, sem, i + _NSLOT, slot)


@functools.partial(jax.jit, static_argnames=())
def kernel(h, W, b):
    n, d = h.shape
    ne = W.shape[0]
    b2 = b.reshape(1, ne)
    grid = (n // _SUPER,)
    return pl.pallas_call(
        _gate_kernel,
        grid=grid,
        in_specs=[
            pl.BlockSpec(memory_space=pl.ANY),
            pl.BlockSpec((ne, d), lambda i: (0, 0)),
            pl.BlockSpec((1, ne), lambda i: (0, 0)),
        ],
        out_specs=pl.BlockSpec((_SUPER, ne), lambda i: (i, 0)),
        out_shape=jax.ShapeDtypeStruct((n, ne), jnp.float32),
        scratch_shapes=[
            pltpu.VMEM((_NSLOT, _SUPER, 2048), jnp.float32),
            pltpu.SemaphoreType.DMA((_NSLOT, _NSUB)),
        ],
        compiler_params=pltpu.CompilerParams(
            dimension_semantics=("arbitrary",),
        ),
    )(h, W, b2)
